# Initial kernel scaffold; baseline (speedup 1.0000x reference)
import jax
import jax.numpy as jnp
from jax.experimental import pallas as pl


def kernel(x, x_lap_pos_enc, edge_index0, edge_index1, node_ids, src_nodes, dst_nodes, emb_h_W, emb_h_b, emb_lap_W, emb_lap_b, Wq, bq, Wk, bk, Wv, bv, Wo, bo, ln1_g, ln1_b, ffn_W1, ffn_b1, ffn_W2, ffn_b2, ln2_g, ln2_b, mlp_W0, mlp_b0, mlp_W1, mlp_b1, mlp_W2, mlp_b2):
    raise NotImplementedError("write your pallas kernel here")


# speedup vs baseline: 1.0025x; 1.0025x over previous
import functools

import jax
import jax.numpy as jnp
import numpy as np
from jax.experimental import pallas as pl

N = 10000
E = 320000
HID = 128
H = 8
DH = 16
NC_OUT = 40


def _ln(x, g, b):
    mu = jnp.mean(x, axis=-1, keepdims=True)
    var = jnp.var(x, axis=-1, keepdims=True)
    return (x - mu) / jnp.sqrt(var + 1e-5) * g + b


def _copy_kernel(x_ref, o_ref):
    o_ref[...] = x_ref[...]


def _gt_layer(ei, h_src, h_dst, p):
    Wq, bq, Wk, bk, Wv, bv, Wo, bo, g1, b1, W1, c1, W2, c2, g2, b2 = p
    Nd = h_dst.shape[0]
    dh = HID // H
    Q = (h_dst @ Wq + bq).reshape(Nd, H, dh)
    K = (h_src @ Wk + bk).reshape(-1, H, dh)
    V = (h_src @ Wv + bv).reshape(-1, H, dh)
    s, d = ei[0], ei[1]
    sc = jnp.sum(K[s] * Q[d], axis=-1) / float(np.sqrt(dh))
    m = jax.ops.segment_max(sc, d, num_segments=Nd)
    m = jnp.where(jnp.isfinite(m), m, 0.0)
    ex = jnp.exp(sc - m[d])
    den = jax.ops.segment_sum(ex, d, num_segments=Nd) + 1e-9
    attn = ex / den[d]
    msg = attn[:, :, None] * V[s]
    h_attn = jax.ops.segment_sum(msg, d, num_segments=Nd).reshape(Nd, HID)
    h = h_attn @ Wo + bo
    h = _ln(h_dst + h, g1, b1)
    h2 = jax.nn.relu(h @ W1 + c1) @ W2 + c2
    return _ln(h + h2, g2, b2)


def kernel(x, x_lap_pos_enc, edge_index0, edge_index1, node_ids, src_nodes, dst_nodes,
           emb_h_W, emb_h_b, emb_lap_W, emb_lap_b,
           Wq, bq, Wk, bk, Wv, bv, Wo, bo, ln1_g, ln1_b,
           ffn_W1, ffn_b1, ffn_W2, ffn_b2, ln2_g, ln2_b,
           mlp_W0, mlp_b0, mlp_W1, mlp_b1, mlp_W2, mlp_b2):
    x = pl.pallas_call(
        _copy_kernel,
        out_shape=jax.ShapeDtypeStruct(x.shape, x.dtype),
    )(x)
    h_lap = x_lap_pos_enc @ emb_lap_W + emb_lap_b
    h0 = x @ emb_h_W + emb_h_b + h_lap
    h_src = h0
    eis = [edge_index0, edge_index1]
    for i in range(2):
        p = (Wq[i], bq[i], Wk[i], bk[i], Wv[i], bv[i], Wo[i], bo[i],
             ln1_g[i], ln1_b[i], ffn_W1[i], ffn_b1[i], ffn_W2[i], ffn_b2[i],
             ln2_g[i], ln2_b[i])
        h_src = _gt_layer(eis[i], h_src, h0, p)
    h = jax.nn.relu(h_src @ mlp_W0 + mlp_b0)
    h = jax.nn.relu(h @ mlp_W1 + mlp_b1)
    return h @ mlp_W2 + mlp_b2


# speedup vs baseline: 29.2350x; 29.1613x over previous
import functools

import jax
import jax.numpy as jnp
import numpy as np
from jax import lax
from jax.experimental import pallas as pl
from jax.experimental.pallas import tpu as pltpu
from jax.experimental.pallas import tpu_sc as plsc

N = 10000
E = 320000
HID = 128
H = 8
DH = 16
NC_OUT = 40

NCORES = 2
NSUB = 16
NTILES = NCORES * NSUB
EB = 64
E_PAD = ((E + NTILES * EB - 1) // (NTILES * EB)) * (NTILES * EB)
PER_TILE = E_PAD // NTILES
NB = PER_TILE // EB
TRASH = N + 8
NROWS = 10016
DBASE = NROWS
DROWS = NROWS // 8
TOTAL_ROWS = 11392
SUBROWS = TOTAL_ROWS // NSUB
RB = 1000
GRID = N // RB

_f32 = jnp.float32


def _mesh():
    return plsc.VectorSubcoreMesh(core_axis_name="c", subcore_axis_name="s")


@functools.partial(
    pl.kernel,
    out_type=[
        jax.ShapeDtypeStruct((E_PAD * 16,), _f32),
        jax.ShapeDtypeStruct((NTILES * 16,), _f32),
    ],
    mesh=_mesh(),
    scratch_types=[
        pltpu.VMEM((EB,), jnp.int32),
        pltpu.VMEM((EB,), jnp.int32),
        pltpu.VMEM((EB, HID), _f32),
        pltpu.VMEM((EB, HID), _f32),
        pltpu.VMEM((EB * 16,), _f32),
        pltpu.VMEM((16,), _f32),
        pltpu.SemaphoreType.DMA,
        pltpu.SemaphoreType.DMA,
    ],
    compiler_params=pltpu.CompilerParams(needs_layout_passes=False),
)
def _sc_score(s_hbm, d_hbm, k_hbm, q_hbm, sc_out, mx_out,
              sidx, didx, krows, qrows, scb, mxb, sem1, sem2):
    wid = lax.axis_index("s") * NCORES + lax.axis_index("c")
    base = wid * PER_TILE
    mxb[...] = jnp.zeros((16,), _f32)
    lane = lax.iota(jnp.int32, 16)

    def blk(b, _):
        off = base + b * EB
        pltpu.sync_copy(s_hbm.at[pl.ds(off, EB)], sidx)
        pltpu.sync_copy(d_hbm.at[pl.ds(off, EB)], didx)
        cp1 = pltpu.async_copy(k_hbm.at[sidx], krows, sem1)
        cp2 = pltpu.async_copy(q_hbm.at[didx], qrows, sem2)
        cp1.wait()
        cp2.wait()

        def edge(e, _):
            acc = jnp.zeros((16,), _f32)
            for h in range(H):
                kv = krows[e, pl.ds(h * DH, DH)]
                qv = qrows[e, pl.ds(h * DH, DH)]
                sh = jnp.sum(kv * qv) * 0.25
                acc = jnp.where(lane == h, sh, acc)
            scb[pl.ds(e * 16, 16)] = acc
            mxb[...] = jnp.maximum(mxb[...], acc)
            return 0

        lax.fori_loop(0, EB, edge, 0)
        pltpu.sync_copy(scb, sc_out.at[pl.ds(off * 16, EB * 16)])
        return 0

    lax.fori_loop(0, NB, blk, 0)
    mxb[...] = jnp.full((16,), jnp.max(mxb[...]), _f32)
    pltpu.sync_copy(mxb, mx_out.at[pl.ds(wid * 16, 16)])


@functools.partial(
    pl.kernel,
    out_type=jax.ShapeDtypeStruct((NCORES, TOTAL_ROWS, HID), _f32),
    mesh=_mesh(),
    scratch_types=[
        pltpu.VMEM((EB,), jnp.int32),
        pltpu.VMEM((EB,), jnp.int32),
        pltpu.VMEM((EB,), jnp.int32),
        pltpu.VMEM((EB, HID), _f32),
        pltpu.VMEM((EB * 16,), _f32),
        pltpu.VMEM((EB, HID), _f32),
        pltpu.VMEM((EB, HID), _f32),
        pltpu.VMEM((16,), _f32),
        pltpu.VMEM_SHARED((TOTAL_ROWS, HID), _f32),
        pltpu.SemaphoreType.DMA,
    ],
    compiler_params=pltpu.CompilerParams(needs_layout_passes=False),
)
def _sc_message(s_hbm, d_hbm, sc_hbm, v_hbm, gm_hbm,
                num_out,
                sidx, didx, didx2, vrows, scb, msg, denb, gm,
                num_sp, sem1):
    c = lax.axis_index("c")
    s = lax.axis_index("s")
    wid = s * NCORES + c
    base = wid * PER_TILE
    pltpu.sync_copy(gm_hbm, gm)
    z16 = jnp.zeros((16,), _f32)
    lane = lax.iota(jnp.int32, 16)
    mlo = jnp.where(lane < H, 1.0, 0.0).astype(_f32)

    def zrow(i, _):
        for j in range(HID // 16):
            denb[i, pl.ds(j * 16, 16)] = z16
        return 0

    lax.fori_loop(0, EB, zrow, 0)
    for j in range(SUBROWS // EB):
        pltpu.sync_copy(denb, num_sp.at[pl.ds(s * SUBROWS + j * EB, EB)])
    rem = SUBROWS % EB
    if rem:
        pltpu.sync_copy(denb.at[pl.ds(0, rem)],
                        num_sp.at[pl.ds(s * SUBROWS + (SUBROWS // EB) * EB, rem)])
    plsc.subcore_barrier()

    def blk(b, _):
        off = base + b * EB
        pltpu.sync_copy(s_hbm.at[pl.ds(off, EB)], sidx)
        pltpu.sync_copy(d_hbm.at[pl.ds(off, EB)], didx)
        cp1 = pltpu.async_copy(v_hbm.at[sidx], vrows, sem1)
        pltpu.sync_copy(sc_hbm.at[pl.ds(off * 16, EB * 16)], scb)
        gmv = gm[...]
        for g in range(EB // 16):
            dv = didx[pl.ds(g * 16, 16)]
            didx2[pl.ds(g * 16, 16)] = DBASE + lax.shift_right_logical(dv, 3)
        cp1.wait()

        def grp(g, _):
            dvec = didx[pl.ds(g * 16, 16)]
            for e16 in range(16):
                e = g * 16 + e16
                d_e = dvec[e16]
                exr = jnp.exp(scb[pl.ds(e * 16, 16)] - gmv)
                for h in range(H):
                    msg[e, pl.ds(h * DH, DH)] = exr[h] * vrows[e, pl.ds(h * DH, DH)]
                for j in range(HID // 16):
                    denb[e, pl.ds(j * 16, 16)] = z16
                doff = (d_e & 7) * 16
                denb[e, pl.ds(doff, 16)] = exr * mlo
            return 0

        lax.fori_loop(0, EB // 16, grp, 0)
        pltpu.sync_copy(msg, num_sp.at[didx], add=True)
        pltpu.sync_copy(denb, num_sp.at[didx2], add=True)
        return 0

    lax.fori_loop(0, NB, blk, 0)
    plsc.subcore_barrier()
    pltpu.sync_copy(num_sp.at[pl.ds(s * SUBROWS, SUBROWS)],
                    num_out.at[c, pl.ds(s * SUBROWS, SUBROWS)])


def _full(shape):
    return pl.BlockSpec(shape, lambda i: tuple(0 for _ in shape))


def _rows(depth=None):
    if depth is None:
        return pl.BlockSpec((RB, HID), lambda i: (i, 0))
    return pl.BlockSpec(depth, lambda i: (0, i, 0))


def _mm(a, b):
    return jnp.dot(a, b, preferred_element_type=_f32)


def _ln_tc(x, g, b):
    mu = jnp.mean(x, axis=-1, keepdims=True)
    xc = x - mu
    var = jnp.mean(xc * xc, axis=-1, keepdims=True)
    return xc * lax.rsqrt(var + 1e-5) * g + b


def _embed_qkv_body(x_ref, lap_ref, we_ref, wl_ref, be_ref,
                    wq_ref, bq_ref, wk_ref, bk_ref, wv_ref, bv_ref,
                    h0_ref, q_ref, k_ref, v_ref):
    h0 = _mm(x_ref[...], we_ref[...]) + _mm(lap_ref[...], wl_ref[...]) + be_ref[...]
    h0_ref[...] = h0
    q_ref[...] = _mm(h0, wq_ref[...]) + bq_ref[...]
    k_ref[...] = _mm(h0, wk_ref[...]) + bk_ref[...]
    v_ref[...] = _mm(h0, wv_ref[...]) + bv_ref[...]


def _attn_post(num_ref, den_ref, h0_ref, r16_ref, wo_ref, bo_ref,
               g1_ref, b1_ref, w1_ref, c1_ref, w2_ref, c2_ref,
               g2_ref, b2_ref):
    nm = num_ref[0] + num_ref[1]
    dn = jnp.sum(den_ref[...], axis=0) + 1e-9
    den_exp = _mm(dn, r16_ref[...])
    h_attn = nm / den_exp
    h = _mm(h_attn, wo_ref[...]) + bo_ref[...]
    t = _ln_tc(h0_ref[...] + h, g1_ref[...], b1_ref[...])
    h2 = _mm(jnp.maximum(_mm(t, w1_ref[...]) + c1_ref[...], 0.0), w2_ref[...]) + c2_ref[...]
    return _ln_tc(t + h2, g2_ref[...], b2_ref[...])


def _post_qkv_body(num_ref, den_ref, h0_ref, r16_ref, wo_ref, bo_ref,
                   g1_ref, b1_ref, w1_ref, c1_ref, w2_ref, c2_ref,
                   g2_ref, b2_ref,
                   wq_ref, bq_ref, wk_ref, bk_ref, wv_ref, bv_ref,
                   q_ref, k_ref, v_ref):
    hs = _attn_post(num_ref, den_ref, h0_ref, r16_ref, wo_ref, bo_ref,
                    g1_ref, b1_ref, w1_ref, c1_ref, w2_ref, c2_ref,
                    g2_ref, b2_ref)
    q_ref[...] = _mm(h0_ref[...], wq_ref[...]) + bq_ref[...]
    k_ref[...] = _mm(hs, wk_ref[...]) + bk_ref[...]
    v_ref[...] = _mm(hs, wv_ref[...]) + bv_ref[...]


def _post_readout_body(num_ref, den_ref, h0_ref, r16_ref, wo_ref, bo_ref,
                       g1_ref, b1_ref, w1_ref, c1_ref, w2_ref, c2_ref,
                       g2_ref, b2_ref,
                       w0p_ref, b0p_ref, w1p_ref, b1p_ref, w2p_ref, b2p_ref,
                       out_ref):
    hs = _attn_post(num_ref, den_ref, h0_ref, r16_ref, wo_ref, bo_ref,
                    g1_ref, b1_ref, w1_ref, c1_ref, w2_ref, c2_ref,
                    g2_ref, b2_ref)
    a = jnp.maximum(_mm(hs, w0p_ref[...]) + b0p_ref[...], 0.0)
    bmid = jnp.maximum(_mm(a, w1p_ref[...]) + b1p_ref[...], 0.0)
    out_ref[...] = _mm(bmid, w2p_ref[...]) + b2p_ref[...]


def _row(v):
    return v.reshape(1, -1)


def kernel(x, x_lap_pos_enc, edge_index0, edge_index1, node_ids, src_nodes, dst_nodes,
           emb_h_W, emb_h_b, emb_lap_W, emb_lap_b,
           Wq, bq, Wk, bk, Wv, bv, Wo, bo, ln1_g, ln1_b,
           ffn_W1, ffn_b1, ffn_W2, ffn_b2, ln2_g, ln2_b,
           mlp_W0, mlp_b0, mlp_W1, mlp_b1, mlp_W2, mlp_b2):
    lap_p = jnp.pad(x_lap_pos_enc, ((0, 0), (0, HID - x_lap_pos_enc.shape[1])))
    wl_p = jnp.pad(emb_lap_W, ((0, HID - emb_lap_W.shape[0]), (0, 0)))
    pad_e = E_PAD - E
    edges = []
    for ei in (edge_index0, edge_index1):
        s = jnp.concatenate([ei[0], jnp.zeros((pad_e,), jnp.int32)])
        d = jnp.concatenate([ei[1], jnp.full((pad_e,), TRASH, jnp.int32)])
        edges.append((s, d))
    r16 = np.zeros((16, HID), np.float32)
    for h in range(H):
        r16[h, h * DH:(h + 1) * DH] = 1.0
    r16 = jnp.asarray(r16)
    w0p = jnp.pad(mlp_W0, ((0, 0), (0, HID - mlp_W0.shape[1])))
    b0p = _row(jnp.pad(mlp_b0, (0, HID - mlp_b0.shape[0])))
    w1p = jnp.pad(mlp_W1, ((0, HID - mlp_W1.shape[0]), (0, HID - mlp_W1.shape[1])))
    b1p = _row(jnp.pad(mlp_b1, (0, HID - mlp_b1.shape[0])))
    w2p = jnp.pad(mlp_W2, ((0, HID - mlp_W2.shape[0]), (0, HID - mlp_W2.shape[1])))
    b2p = _row(jnp.pad(mlp_b2, (0, HID - mlp_b2.shape[0])))

    fullw = _full((HID, HID))
    fullb = _full((1, HID))
    w256 = _full((HID, 2 * HID))
    b256 = _full((1, 2 * HID))
    shd = jax.ShapeDtypeStruct((N, HID), _f32)

    h0, q1, k1, v1 = pl.pallas_call(
        _embed_qkv_body,
        grid=(GRID,),
        in_specs=[_rows(), _rows(), fullw, fullw, fullb,
                  fullw, fullb, fullw, fullb, fullw, fullb],
        out_specs=[_rows(), _rows(), _rows(), _rows()],
        out_shape=[shd, shd, shd, shd],
    )(x, lap_p, emb_h_W, wl_p, _row(emb_h_b),
      Wq[0], _row(bq[0]), Wk[0], _row(bk[0]), Wv[0], _row(bv[0]))

    num = den = None
    for i in range(2):
        s_e, d_e = edges[i]
        qq, kk, vv = (q1, k1, v1) if i == 0 else (q2, k2, v2)
        sc_e, mx = _sc_score(s_e, d_e, kk, qq)
        gmax = jnp.maximum(jnp.max(mx), 0.0)
        gm_arr = jnp.full((16,), gmax, _f32)
        num = _sc_message(s_e, d_e, sc_e, vv, gm_arr)
        den = num[:, DBASE:DBASE + DROWS].reshape(NCORES, NROWS, 16)

        post_in = [num, den, h0, r16, Wo[i], _row(bo[i]),
                   _row(ln1_g[i]), _row(ln1_b[i]), ffn_W1[i], _row(ffn_b1[i]),
                   ffn_W2[i], _row(ffn_b2[i]), _row(ln2_g[i]), _row(ln2_b[i])]
        post_specs = [_rows((NCORES, RB, HID)), _rows((NCORES, RB, 16)),
                      _rows(), _full((16, HID)), fullw, fullb,
                      fullb, fullb, w256, b256,
                      _full((2 * HID, HID)), fullb, fullb, fullb]
        if i == 0:
            q2, k2, v2 = pl.pallas_call(
                _post_qkv_body,
                grid=(GRID,),
                in_specs=post_specs + [fullw, fullb, fullw, fullb, fullw, fullb],
                out_specs=[_rows(), _rows(), _rows()],
                out_shape=[shd, shd, shd],
            )(*post_in, Wq[1], _row(bq[1]), Wk[1], _row(bk[1]), Wv[1], _row(bv[1]))
        else:
            logits = pl.pallas_call(
                _post_readout_body,
                grid=(GRID,),
                in_specs=post_specs + [fullw, fullb, fullw, fullb, fullw, fullb],
                out_specs=_rows(),
                out_shape=shd,
            )(*post_in, w0p, b0p, w1p, b1p, w2p, b2p)
    return logits[:, :NC_OUT]


# speedup vs baseline: 33.7476x; 1.1544x over previous
import functools

import jax
import jax.numpy as jnp
import numpy as np
from jax import lax
from jax.experimental import pallas as pl
from jax.experimental.pallas import tpu as pltpu
from jax.experimental.pallas import tpu_sc as plsc

N = 10000
E = 320000
HID = 128
H = 8
DH = 16
NC_OUT = 40

NCORES = 2
NSUB = 16
NTILES = NCORES * NSUB
EBS = 128
EB = 64
E_PAD = ((E + NTILES * EBS - 1) // (NTILES * EBS)) * (NTILES * EBS)
PER_TILE = E_PAD // NTILES
NBS = PER_TILE // EBS
NB = PER_TILE // EB
TRASH = N + 8
NROWS = 10016
DBASE = NROWS
DROWS = NROWS // 8
TOTAL_ROWS = 11392
SUBROWS = TOTAL_ROWS // NSUB
RB = 1000
GRID = N // RB

_f32 = jnp.float32


def _mesh():
    return plsc.VectorSubcoreMesh(core_axis_name="c", subcore_axis_name="s")


@functools.partial(
    pl.kernel,
    out_type=[
        jax.ShapeDtypeStruct((E_PAD * 16,), _f32),
        jax.ShapeDtypeStruct((NTILES * 16,), _f32),
    ],
    mesh=_mesh(),
    scratch_types=[
        pltpu.VMEM((2, EBS), jnp.int32),
        pltpu.VMEM((2, EBS), jnp.int32),
        pltpu.VMEM((2, EBS, HID), _f32),
        pltpu.VMEM((2, EBS, HID), _f32),
        pltpu.VMEM((2, EBS * 16), _f32),
        pltpu.VMEM((16,), _f32),
        pltpu.SemaphoreType.DMA,
        pltpu.SemaphoreType.DMA,
        pltpu.SemaphoreType.DMA,
    ],
    compiler_params=pltpu.CompilerParams(needs_layout_passes=False),
)
def _sc_score(s_hbm, d_hbm, k_hbm, q_hbm, sc_out, mx_out,
              sidx, didx, krows, qrows, scb, mxb, semi, semg, semw):
    wid = lax.axis_index("s") * NCORES + lax.axis_index("c")
    base = wid * PER_TILE
    mxb[...] = jnp.zeros((16,), _f32)
    lane = lax.iota(jnp.int32, 16)

    def idx_copies(b, t):
        off = base + b * EBS
        c1 = pltpu.make_async_copy(s_hbm.at[pl.ds(off, EBS)], sidx.at[t], semi)
        c2 = pltpu.make_async_copy(d_hbm.at[pl.ds(off, EBS)], didx.at[t], semi)
        return c1, c2

    def gathers(t):
        c1 = pltpu.make_async_copy(k_hbm.at[sidx.at[t]], krows.at[t], semg)
        c2 = pltpu.make_async_copy(q_hbm.at[didx.at[t]], qrows.at[t], semg)
        return c1, c2

    for cp in idx_copies(0, 0):
        cp.start()
    for cp in idx_copies(0, 0):
        cp.wait()
    for cp in gathers(0):
        cp.start()
    for cp in idx_copies(1, 1):
        cp.start()

    def blk(b, _):
        t = lax.rem(b, 2)
        tn = 1 - t
        for cp in gathers(t):
            cp.wait()

        @pl.when(b + 1 < NBS)
        def _():
            for cp in idx_copies(b + 1, tn):
                cp.wait()
            for cp in gathers(tn):
                cp.start()

        @pl.when(b + 2 < NBS)
        def _():
            for cp in idx_copies(b + 2, t):
                cp.start()

        @pl.when(b >= 2)
        def _():
            pltpu.make_async_copy(
                scb.at[t], sc_out.at[pl.ds(0, EBS * 16)], semw).wait()

        def edge(e, _):
            acc = jnp.zeros((16,), _f32)
            for h in range(H):
                kv = krows[t, e, pl.ds(h * DH, DH)]
                qv = qrows[t, e, pl.ds(h * DH, DH)]
                sh = jnp.sum(kv * qv) * 0.25
                acc = jnp.where(lane == h, sh, acc)
            scb[t, pl.ds(e * 16, 16)] = acc
            mxb[...] = jnp.maximum(mxb[...], acc)
            return 0

        lax.fori_loop(0, EBS, edge, 0)
        off = base + b * EBS
        pltpu.make_async_copy(
            scb.at[t], sc_out.at[pl.ds(off * 16, EBS * 16)], semw).start()
        return 0

    lax.fori_loop(0, NBS, blk, 0)
    for _ in range(2):
        pltpu.make_async_copy(
            scb.at[0], sc_out.at[pl.ds(0, EBS * 16)], semw).wait()
    mxb[...] = jnp.full((16,), jnp.max(mxb[...]), _f32)
    pltpu.sync_copy(mxb, mx_out.at[pl.ds(wid * 16, 16)])


@functools.partial(
    pl.kernel,
    out_type=jax.ShapeDtypeStruct((NCORES, TOTAL_ROWS, HID), _f32),
    mesh=_mesh(),
    scratch_types=[
        pltpu.VMEM((2, EB), jnp.int32),
        pltpu.VMEM((2, EB), jnp.int32),
        pltpu.VMEM((2, EB), jnp.int32),
        pltpu.VMEM((2, EB), jnp.int32),
        pltpu.VMEM((2, EB, HID), _f32),
        pltpu.VMEM((2, EB * 16), _f32),
        pltpu.VMEM((EB, HID), _f32),
        pltpu.VMEM((EB, HID), _f32),
        pltpu.VMEM((16,), _f32),
        pltpu.VMEM_SHARED((TOTAL_ROWS, HID), _f32),
        pltpu.SemaphoreType.DMA,
        pltpu.SemaphoreType.DMA,
        pltpu.SemaphoreType.DMA,
    ],
    compiler_params=pltpu.CompilerParams(needs_layout_passes=False),
)
def _sc_message(s_hbm, d_hbm, sc_hbm, v_hbm, gm_hbm,
                num_out,
                sidx, didx, didx_sc, didx2, vrows, scb, msg, denb, gm,
                num_sp, semi, semg, semw):
    c = lax.axis_index("c")
    s = lax.axis_index("s")
    wid = s * NCORES + c
    base = wid * PER_TILE
    pltpu.sync_copy(gm_hbm, gm)
    z16 = jnp.zeros((16,), _f32)
    lane = lax.iota(jnp.int32, 16)
    mlo = jnp.where(lane < H, 1.0, 0.0).astype(_f32)

    def zrow(i, _):
        for j in range(HID // 16):
            denb[i, pl.ds(j * 16, 16)] = z16
        return 0

    lax.fori_loop(0, EB, zrow, 0)
    for j in range(SUBROWS // EB):
        pltpu.sync_copy(denb, num_sp.at[pl.ds(s * SUBROWS + j * EB, EB)])
    rem = SUBROWS % EB
    if rem:
        pltpu.sync_copy(denb.at[pl.ds(0, rem)],
                        num_sp.at[pl.ds(s * SUBROWS + (SUBROWS // EB) * EB, rem)])
    plsc.subcore_barrier()

    def idx_copies(b, t):
        off = base + b * EB
        c1 = pltpu.make_async_copy(s_hbm.at[pl.ds(off, EB)], sidx.at[t], semi)
        c2 = pltpu.make_async_copy(d_hbm.at[pl.ds(off, EB)], didx.at[t], semi)
        return c1, c2

    def gathers(b, t):
        off = base + b * EB
        c1 = pltpu.make_async_copy(v_hbm.at[sidx.at[t]], vrows.at[t], semg)
        c2 = pltpu.make_async_copy(
            sc_hbm.at[pl.ds(off * 16, EB * 16)], scb.at[t], semg)
        return c1, c2

    def scatter_drains():
        c1 = pltpu.make_async_copy(msg, num_sp.at[pl.ds(0, EB)], semw)
        c2 = pltpu.make_async_copy(denb, num_sp.at[pl.ds(0, EB)], semw)
        return c1, c2

    for cp in idx_copies(0, 0):
        cp.start()
    for cp in idx_copies(0, 0):
        cp.wait()
    for cp in gathers(0, 0):
        cp.start()
    for cp in idx_copies(1, 1):
        cp.start()

    def blk(b, _):
        t = lax.rem(b, 2)
        tn = 1 - t
        for cp in gathers(b, t):
            cp.wait()

        @pl.when(b + 1 < NB)
        def _():
            for cp in idx_copies(b + 1, tn):
                cp.wait()
            for cp in gathers(b + 1, tn):
                cp.start()

        for g in range(EB // 16):
            dv = didx[t, pl.ds(g * 16, 16)]
            didx_sc[t, pl.ds(g * 16, 16)] = dv
            didx2[t, pl.ds(g * 16, 16)] = DBASE + lax.shift_right_logical(dv, 3)

        @pl.when(b + 2 < NB)
        def _():
            for cp in idx_copies(b + 2, t):
                cp.start()

        @pl.when(b >= 1)
        def _():
            for cp in scatter_drains():
                cp.wait()

        gmv = gm[...]

        def grp(g, _):
            dvec = didx_sc[t, pl.ds(g * 16, 16)]
            for e16 in range(16):
                e = g * 16 + e16
                d_e = dvec[e16]
                exr = jnp.exp(scb[t, pl.ds(e * 16, 16)] - gmv)
                for h in range(H):
                    msg[e, pl.ds(h * DH, DH)] = exr[h] * vrows[t, e, pl.ds(h * DH, DH)]
                for j in range(HID // 16):
                    denb[e, pl.ds(j * 16, 16)] = z16
                doff = (d_e & 7) * 16
                denb[e, pl.ds(doff, 16)] = exr * mlo
            return 0

        lax.fori_loop(0, EB // 16, grp, 0)
        pltpu.async_copy(msg, num_sp.at[didx_sc.at[t]], semw, add=True)
        pltpu.async_copy(denb, num_sp.at[didx2.at[t]], semw, add=True)
        return 0

    lax.fori_loop(0, NB, blk, 0)
    for cp in scatter_drains():
        cp.wait()
    plsc.subcore_barrier()
    pltpu.sync_copy(num_sp.at[pl.ds(s * SUBROWS, SUBROWS)],
                    num_out.at[c, pl.ds(s * SUBROWS, SUBROWS)])


def _full(shape):
    return pl.BlockSpec(shape, lambda i: tuple(0 for _ in shape))


def _rows(depth=None):
    if depth is None:
        return pl.BlockSpec((RB, HID), lambda i: (i, 0))
    return pl.BlockSpec(depth, lambda i: (0, i, 0))


def _mm(a, b):
    return jnp.dot(a, b, preferred_element_type=_f32)


def _ln_tc(x, g, b):
    mu = jnp.mean(x, axis=-1, keepdims=True)
    xc = x - mu
    var = jnp.mean(xc * xc, axis=-1, keepdims=True)
    return xc * lax.rsqrt(var + 1e-5) * g + b


def _embed_qkv_body(x_ref, lap_ref, we_ref, wl_ref, be_ref,
                    wq_ref, bq_ref, wk_ref, bk_ref, wv_ref, bv_ref,
                    h0_ref, q_ref, k_ref, v_ref):
    h0 = _mm(x_ref[...], we_ref[...]) + _mm(lap_ref[...], wl_ref[...]) + be_ref[...]
    h0_ref[...] = h0
    q_ref[...] = _mm(h0, wq_ref[...]) + bq_ref[...]
    k_ref[...] = _mm(h0, wk_ref[...]) + bk_ref[...]
    v_ref[...] = _mm(h0, wv_ref[...]) + bv_ref[...]


def _attn_post(num_ref, den_ref, h0_ref, r16_ref, wo_ref, bo_ref,
               g1_ref, b1_ref, w1_ref, c1_ref, w2_ref, c2_ref,
               g2_ref, b2_ref):
    nm = num_ref[0] + num_ref[1]
    dn = jnp.sum(den_ref[...], axis=0) + 1e-9
    den_exp = _mm(dn, r16_ref[...])
    h_attn = nm / den_exp
    h = _mm(h_attn, wo_ref[...]) + bo_ref[...]
    t = _ln_tc(h0_ref[...] + h, g1_ref[...], b1_ref[...])
    h2 = _mm(jnp.maximum(_mm(t, w1_ref[...]) + c1_ref[...], 0.0), w2_ref[...]) + c2_ref[...]
    return _ln_tc(t + h2, g2_ref[...], b2_ref[...])


def _post_qkv_body(num_ref, den_ref, h0_ref, r16_ref, wo_ref, bo_ref,
                   g1_ref, b1_ref, w1_ref, c1_ref, w2_ref, c2_ref,
                   g2_ref, b2_ref,
                   wq_ref, bq_ref, wk_ref, bk_ref, wv_ref, bv_ref,
                   q_ref, k_ref, v_ref):
    hs = _attn_post(num_ref, den_ref, h0_ref, r16_ref, wo_ref, bo_ref,
                    g1_ref, b1_ref, w1_ref, c1_ref, w2_ref, c2_ref,
                    g2_ref, b2_ref)
    q_ref[...] = _mm(h0_ref[...], wq_ref[...]) + bq_ref[...]
    k_ref[...] = _mm(hs, wk_ref[...]) + bk_ref[...]
    v_ref[...] = _mm(hs, wv_ref[...]) + bv_ref[...]


def _post_readout_body(num_ref, den_ref, h0_ref, r16_ref, wo_ref, bo_ref,
                       g1_ref, b1_ref, w1_ref, c1_ref, w2_ref, c2_ref,
                       g2_ref, b2_ref,
                       w0p_ref, b0p_ref, w1p_ref, b1p_ref, w2p_ref, b2p_ref,
                       out_ref):
    hs = _attn_post(num_ref, den_ref, h0_ref, r16_ref, wo_ref, bo_ref,
                    g1_ref, b1_ref, w1_ref, c1_ref, w2_ref, c2_ref,
                    g2_ref, b2_ref)
    a = jnp.maximum(_mm(hs, w0p_ref[...]) + b0p_ref[...], 0.0)
    bmid = jnp.maximum(_mm(a, w1p_ref[...]) + b1p_ref[...], 0.0)
    out_ref[...] = _mm(bmid, w2p_ref[...]) + b2p_ref[...]


def _row(v):
    return v.reshape(1, -1)


def kernel(x, x_lap_pos_enc, edge_index0, edge_index1, node_ids, src_nodes, dst_nodes,
           emb_h_W, emb_h_b, emb_lap_W, emb_lap_b,
           Wq, bq, Wk, bk, Wv, bv, Wo, bo, ln1_g, ln1_b,
           ffn_W1, ffn_b1, ffn_W2, ffn_b2, ln2_g, ln2_b,
           mlp_W0, mlp_b0, mlp_W1, mlp_b1, mlp_W2, mlp_b2):
    lap_p = jnp.pad(x_lap_pos_enc, ((0, 0), (0, HID - x_lap_pos_enc.shape[1])))
    wl_p = jnp.pad(emb_lap_W, ((0, HID - emb_lap_W.shape[0]), (0, 0)))
    pad_e = E_PAD - E
    edges = []
    for ei in (edge_index0, edge_index1):
        s = jnp.concatenate([ei[0], jnp.zeros((pad_e,), jnp.int32)])
        d = jnp.concatenate([ei[1], jnp.full((pad_e,), TRASH, jnp.int32)])
        edges.append((s, d))
    r16 = np.zeros((16, HID), np.float32)
    for h in range(H):
        r16[h, h * DH:(h + 1) * DH] = 1.0
    r16 = jnp.asarray(r16)
    w0p = jnp.pad(mlp_W0, ((0, 0), (0, HID - mlp_W0.shape[1])))
    b0p = _row(jnp.pad(mlp_b0, (0, HID - mlp_b0.shape[0])))
    w1p = jnp.pad(mlp_W1, ((0, HID - mlp_W1.shape[0]), (0, HID - mlp_W1.shape[1])))
    b1p = _row(jnp.pad(mlp_b1, (0, HID - mlp_b1.shape[0])))
    w2p = jnp.pad(mlp_W2, ((0, HID - mlp_W2.shape[0]), (0, HID - mlp_W2.shape[1])))
    b2p = _row(jnp.pad(mlp_b2, (0, HID - mlp_b2.shape[0])))

    fullw = _full((HID, HID))
    fullb = _full((1, HID))
    w256 = _full((HID, 2 * HID))
    b256 = _full((1, 2 * HID))
    shd = jax.ShapeDtypeStruct((N, HID), _f32)

    h0, q1, k1, v1 = pl.pallas_call(
        _embed_qkv_body,
        grid=(GRID,),
        in_specs=[_rows(), _rows(), fullw, fullw, fullb,
                  fullw, fullb, fullw, fullb, fullw, fullb],
        out_specs=[_rows(), _rows(), _rows(), _rows()],
        out_shape=[shd, shd, shd, shd],
    )(x, lap_p, emb_h_W, wl_p, _row(emb_h_b),
      Wq[0], _row(bq[0]), Wk[0], _row(bk[0]), Wv[0], _row(bv[0]))

    num = den = None
    for i in range(2):
        s_e, d_e = edges[i]
        qq, kk, vv = (q1, k1, v1) if i == 0 else (q2, k2, v2)
        sc_e, mx = _sc_score(s_e, d_e, kk, qq)
        gmax = jnp.maximum(jnp.max(mx), 0.0)
        gm_arr = jnp.full((16,), gmax, _f32)
        num = _sc_message(s_e, d_e, sc_e, vv, gm_arr)
        den = num[:, DBASE:DBASE + DROWS].reshape(NCORES, NROWS, 16)

        post_in = [num, den, h0, r16, Wo[i], _row(bo[i]),
                   _row(ln1_g[i]), _row(ln1_b[i]), ffn_W1[i], _row(ffn_b1[i]),
                   ffn_W2[i], _row(ffn_b2[i]), _row(ln2_g[i]), _row(ln2_b[i])]
        post_specs = [_rows((NCORES, RB, HID)), _rows((NCORES, RB, 16)),
                      _rows(), _full((16, HID)), fullw, fullb,
                      fullb, fullb, w256, b256,
                      _full((2 * HID, HID)), fullb, fullb, fullb]
        if i == 0:
            q2, k2, v2 = pl.pallas_call(
                _post_qkv_body,
                grid=(GRID,),
                in_specs=post_specs + [fullw, fullb, fullw, fullb, fullw, fullb],
                out_specs=[_rows(), _rows(), _rows()],
                out_shape=[shd, shd, shd],
            )(*post_in, Wq[1], _row(bq[1]), Wk[1], _row(bk[1]), Wv[1], _row(bv[1]))
        else:
            logits = pl.pallas_call(
                _post_readout_body,
                grid=(GRID,),
                in_specs=post_specs + [fullw, fullb, fullw, fullb, fullw, fullb],
                out_specs=_rows(),
                out_shape=shd,
            )(*post_in, w0p, b0p, w1p, b1p, w2p, b2p)
    return logits[:, :NC_OUT]


# speedup vs baseline: 36.8577x; 1.0922x over previous
import functools

import jax
import jax.numpy as jnp
import numpy as np
from jax import lax
from jax.experimental import pallas as pl
from jax.experimental.pallas import tpu as pltpu
from jax.experimental.pallas import tpu_sc as plsc

N = 10000
E = 320000
HID = 128
H = 8
DH = 16
NC_OUT = 40

NCORES = 2
NSUB = 16
NTILES = NCORES * NSUB
EBS = 128
EB = 64
E_PAD = ((E + NTILES * EBS - 1) // (NTILES * EBS)) * (NTILES * EBS)
PER_TILE = E_PAD // NTILES
NBS = PER_TILE // EBS
NB = PER_TILE // EB
TRASH = N + 8
NROWS = 10016
DBASE = NROWS
DROWS = NROWS // 8
TOTAL_ROWS = 11392
SUBROWS = TOTAL_ROWS // NSUB
RB = 1000
GRID = N // RB

_f32 = jnp.float32


def _mesh():
    return plsc.VectorSubcoreMesh(core_axis_name="c", subcore_axis_name="s")


@functools.partial(
    pl.kernel,
    out_type=[
        jax.ShapeDtypeStruct((E_PAD * 16,), _f32),
        jax.ShapeDtypeStruct((NTILES * 16,), _f32),
    ],
    mesh=_mesh(),
    scratch_types=[
        pltpu.VMEM((2, EBS), jnp.int32),
        pltpu.VMEM((2, EBS), jnp.int32),
        pltpu.VMEM((2, EBS, HID), _f32),
        pltpu.VMEM((2, EBS, HID), _f32),
        pltpu.VMEM((2, EBS * 16), _f32),
        pltpu.VMEM((16,), _f32),
        pltpu.SemaphoreType.DMA,
        pltpu.SemaphoreType.DMA,
        pltpu.SemaphoreType.DMA,
    ],
    compiler_params=pltpu.CompilerParams(needs_layout_passes=False),
)
def _sc_score(s_hbm, d_hbm, k_hbm, q_hbm, sc_out, mx_out,
              sidx, didx, krows, qrows, scb, mxb, semi, semg, semw):
    wid = lax.axis_index("s") * NCORES + lax.axis_index("c")
    base = wid * PER_TILE
    mxb[...] = jnp.zeros((16,), _f32)
    lane = lax.iota(jnp.int32, 16)

    def idx_copies(b, t):
        off = base + b * EBS
        c1 = pltpu.make_async_copy(s_hbm.at[pl.ds(off, EBS)], sidx.at[t], semi)
        c2 = pltpu.make_async_copy(d_hbm.at[pl.ds(off, EBS)], didx.at[t], semi)
        return c1, c2

    def gathers(t):
        c1 = pltpu.make_async_copy(k_hbm.at[sidx.at[t]], krows.at[t], semg)
        c2 = pltpu.make_async_copy(q_hbm.at[didx.at[t]], qrows.at[t], semg)
        return c1, c2

    for cp in idx_copies(0, 0):
        cp.start()
    for cp in idx_copies(0, 0):
        cp.wait()
    for cp in gathers(0):
        cp.start()
    for cp in idx_copies(1, 1):
        cp.start()

    def blk(b, _):
        t = lax.rem(b, 2)
        tn = 1 - t
        for cp in gathers(t):
            cp.wait()

        @pl.when(b + 1 < NBS)
        def _():
            for cp in idx_copies(b + 1, tn):
                cp.wait()
            for cp in gathers(tn):
                cp.start()

        @pl.when(b + 2 < NBS)
        def _():
            for cp in idx_copies(b + 2, t):
                cp.start()

        @pl.when(b >= 2)
        def _():
            pltpu.make_async_copy(
                scb.at[t], sc_out.at[pl.ds(0, EBS * 16)], semw).wait()

        @plsc.parallel_loop(0, EBS, carry=jnp.zeros((16,), _f32))
        def mx_final(e, mx):
            acc = jnp.zeros((16,), _f32)
            for h in range(H):
                kv = krows[t, e, pl.ds(h * DH, DH)]
                qv = qrows[t, e, pl.ds(h * DH, DH)]
                sh = jnp.sum(kv * qv) * 0.25
                acc = jnp.where(lane == h, sh, acc)
            scb[t, pl.ds(e * 16, 16)] = acc
            return jnp.maximum(mx, acc)

        mxb[...] = jnp.maximum(mxb[...], mx_final)
        off = base + b * EBS
        pltpu.make_async_copy(
            scb.at[t], sc_out.at[pl.ds(off * 16, EBS * 16)], semw).start()
        return 0

    lax.fori_loop(0, NBS, blk, 0)
    for _ in range(2):
        pltpu.make_async_copy(
            scb.at[0], sc_out.at[pl.ds(0, EBS * 16)], semw).wait()
    mxb[...] = jnp.full((16,), jnp.max(mxb[...]), _f32)
    pltpu.sync_copy(mxb, mx_out.at[pl.ds(wid * 16, 16)])


@functools.partial(
    pl.kernel,
    out_type=jax.ShapeDtypeStruct((NCORES, TOTAL_ROWS, HID), _f32),
    mesh=_mesh(),
    scratch_types=[
        pltpu.VMEM((2, EB), jnp.int32),
        pltpu.VMEM((2, EB), jnp.int32),
        pltpu.VMEM((2, EB), jnp.int32),
        pltpu.VMEM((2, EB), jnp.int32),
        pltpu.VMEM((2, EB, HID), _f32),
        pltpu.VMEM((2, EB * 16), _f32),
        pltpu.VMEM((EB, HID), _f32),
        pltpu.VMEM((EB, HID), _f32),
        pltpu.VMEM((16,), _f32),
        pltpu.VMEM_SHARED((TOTAL_ROWS, HID), _f32),
        pltpu.SemaphoreType.DMA,
        pltpu.SemaphoreType.DMA,
        pltpu.SemaphoreType.DMA,
    ],
    compiler_params=pltpu.CompilerParams(needs_layout_passes=False),
)
def _sc_message(s_hbm, d_hbm, sc_hbm, v_hbm, gm_hbm,
                num_out,
                sidx, didx, didx_sc, didx2, vrows, scb, msg, denb, gm,
                num_sp, semi, semg, semw):
    c = lax.axis_index("c")
    s = lax.axis_index("s")
    wid = s * NCORES + c
    base = wid * PER_TILE
    pltpu.sync_copy(gm_hbm, gm)
    z16 = jnp.zeros((16,), _f32)
    lane = lax.iota(jnp.int32, 16)
    mlo = jnp.where(lane < H, 1.0, 0.0).astype(_f32)

    def zrow(i, _):
        for j in range(HID // 16):
            denb[i, pl.ds(j * 16, 16)] = z16
        return 0

    lax.fori_loop(0, EB, zrow, 0)
    for j in range(SUBROWS // EB):
        pltpu.sync_copy(denb, num_sp.at[pl.ds(s * SUBROWS + j * EB, EB)])
    rem = SUBROWS % EB
    if rem:
        pltpu.sync_copy(denb.at[pl.ds(0, rem)],
                        num_sp.at[pl.ds(s * SUBROWS + (SUBROWS // EB) * EB, rem)])
    plsc.subcore_barrier()

    def idx_copies(b, t):
        off = base + b * EB
        c1 = pltpu.make_async_copy(s_hbm.at[pl.ds(off, EB)], sidx.at[t], semi)
        c2 = pltpu.make_async_copy(d_hbm.at[pl.ds(off, EB)], didx.at[t], semi)
        return c1, c2

    def gathers(b, t):
        off = base + b * EB
        c1 = pltpu.make_async_copy(v_hbm.at[sidx.at[t]], vrows.at[t], semg)
        c2 = pltpu.make_async_copy(
            sc_hbm.at[pl.ds(off * 16, EB * 16)], scb.at[t], semg)
        return c1, c2

    def scatter_drains():
        c1 = pltpu.make_async_copy(msg, num_sp.at[pl.ds(0, EB)], semw)
        c2 = pltpu.make_async_copy(denb, num_sp.at[pl.ds(0, EB)], semw)
        return c1, c2

    for cp in idx_copies(0, 0):
        cp.start()
    for cp in idx_copies(0, 0):
        cp.wait()
    for cp in gathers(0, 0):
        cp.start()
    for cp in idx_copies(1, 1):
        cp.start()

    def blk(b, _):
        t = lax.rem(b, 2)
        tn = 1 - t
        for cp in gathers(b, t):
            cp.wait()

        @pl.when(b + 1 < NB)
        def _():
            for cp in idx_copies(b + 1, tn):
                cp.wait()
            for cp in gathers(b + 1, tn):
                cp.start()

        for g in range(EB // 16):
            dv = didx[t, pl.ds(g * 16, 16)]
            didx_sc[t, pl.ds(g * 16, 16)] = dv
            didx2[t, pl.ds(g * 16, 16)] = DBASE + lax.shift_right_logical(dv, 3)

        @pl.when(b + 2 < NB)
        def _():
            for cp in idx_copies(b + 2, t):
                cp.start()

        @pl.when(b >= 1)
        def _():
            for cp in scatter_drains():
                cp.wait()

        gmv = gm[...]

        @plsc.parallel_loop(0, EB // 16)
        def _(g):
            dvec = didx_sc[t, pl.ds(g * 16, 16)]
            for e16 in range(16):
                e = g * 16 + e16
                d_e = dvec[e16]
                exr = jnp.exp(scb[t, pl.ds(e * 16, 16)] - gmv)
                for h in range(H):
                    msg[e, pl.ds(h * DH, DH)] = exr[h] * vrows[t, e, pl.ds(h * DH, DH)]
                for j in range(HID // 16):
                    denb[e, pl.ds(j * 16, 16)] = z16
                doff = (d_e & 7) * 16
                denb[e, pl.ds(doff, 16)] = exr * mlo
        pltpu.async_copy(msg, num_sp.at[didx_sc.at[t]], semw, add=True)
        pltpu.async_copy(denb, num_sp.at[didx2.at[t]], semw, add=True)
        return 0

    lax.fori_loop(0, NB, blk, 0)
    for cp in scatter_drains():
        cp.wait()
    plsc.subcore_barrier()
    pltpu.sync_copy(num_sp.at[pl.ds(s * SUBROWS, SUBROWS)],
                    num_out.at[c, pl.ds(s * SUBROWS, SUBROWS)])


def _full(shape):
    return pl.BlockSpec(shape, lambda i: tuple(0 for _ in shape))


def _rows(depth=None):
    if depth is None:
        return pl.BlockSpec((RB, HID), lambda i: (i, 0))
    return pl.BlockSpec(depth, lambda i: (0, i, 0))


def _mm(a, b):
    return jnp.dot(a, b, preferred_element_type=_f32)


def _ln_tc(x, g, b):
    mu = jnp.mean(x, axis=-1, keepdims=True)
    xc = x - mu
    var = jnp.mean(xc * xc, axis=-1, keepdims=True)
    return xc * lax.rsqrt(var + 1e-5) * g + b


def _embed_qkv_body(x_ref, lap_ref, we_ref, wl_ref, be_ref,
                    wq_ref, bq_ref, wk_ref, bk_ref, wv_ref, bv_ref,
                    h0_ref, q_ref, k_ref, v_ref):
    h0 = _mm(x_ref[...], we_ref[...]) + _mm(lap_ref[...], wl_ref[...]) + be_ref[...]
    h0_ref[...] = h0
    q_ref[...] = _mm(h0, wq_ref[...]) + bq_ref[...]
    k_ref[...] = _mm(h0, wk_ref[...]) + bk_ref[...]
    v_ref[...] = _mm(h0, wv_ref[...]) + bv_ref[...]


def _attn_post(num_ref, den_ref, h0_ref, r16_ref, wo_ref, bo_ref,
               g1_ref, b1_ref, w1_ref, c1_ref, w2_ref, c2_ref,
               g2_ref, b2_ref):
    nm = num_ref[0] + num_ref[1]
    dn = jnp.sum(den_ref[...], axis=0) + 1e-9
    den_exp = _mm(dn, r16_ref[...])
    h_attn = nm / den_exp
    h = _mm(h_attn, wo_ref[...]) + bo_ref[...]
    t = _ln_tc(h0_ref[...] + h, g1_ref[...], b1_ref[...])
    h2 = _mm(jnp.maximum(_mm(t, w1_ref[...]) + c1_ref[...], 0.0), w2_ref[...]) + c2_ref[...]
    return _ln_tc(t + h2, g2_ref[...], b2_ref[...])


def _post_qkv_body(num_ref, den_ref, h0_ref, r16_ref, wo_ref, bo_ref,
                   g1_ref, b1_ref, w1_ref, c1_ref, w2_ref, c2_ref,
                   g2_ref, b2_ref,
                   wq_ref, bq_ref, wk_ref, bk_ref, wv_ref, bv_ref,
                   q_ref, k_ref, v_ref):
    hs = _attn_post(num_ref, den_ref, h0_ref, r16_ref, wo_ref, bo_ref,
                    g1_ref, b1_ref, w1_ref, c1_ref, w2_ref, c2_ref,
                    g2_ref, b2_ref)
    q_ref[...] = _mm(h0_ref[...], wq_ref[...]) + bq_ref[...]
    k_ref[...] = _mm(hs, wk_ref[...]) + bk_ref[...]
    v_ref[...] = _mm(hs, wv_ref[...]) + bv_ref[...]


def _post_readout_body(num_ref, den_ref, h0_ref, r16_ref, wo_ref, bo_ref,
                       g1_ref, b1_ref, w1_ref, c1_ref, w2_ref, c2_ref,
                       g2_ref, b2_ref,
                       w0p_ref, b0p_ref, w1p_ref, b1p_ref, w2p_ref, b2p_ref,
                       out_ref):
    hs = _attn_post(num_ref, den_ref, h0_ref, r16_ref, wo_ref, bo_ref,
                    g1_ref, b1_ref, w1_ref, c1_ref, w2_ref, c2_ref,
                    g2_ref, b2_ref)
    a = jnp.maximum(_mm(hs, w0p_ref[...]) + b0p_ref[...], 0.0)
    bmid = jnp.maximum(_mm(a, w1p_ref[...]) + b1p_ref[...], 0.0)
    out_ref[...] = _mm(bmid, w2p_ref[...]) + b2p_ref[...]


def _row(v):
    return v.reshape(1, -1)


def kernel(x, x_lap_pos_enc, edge_index0, edge_index1, node_ids, src_nodes, dst_nodes,
           emb_h_W, emb_h_b, emb_lap_W, emb_lap_b,
           Wq, bq, Wk, bk, Wv, bv, Wo, bo, ln1_g, ln1_b,
           ffn_W1, ffn_b1, ffn_W2, ffn_b2, ln2_g, ln2_b,
           mlp_W0, mlp_b0, mlp_W1, mlp_b1, mlp_W2, mlp_b2):
    lap_p = jnp.pad(x_lap_pos_enc, ((0, 0), (0, HID - x_lap_pos_enc.shape[1])))
    wl_p = jnp.pad(emb_lap_W, ((0, HID - emb_lap_W.shape[0]), (0, 0)))
    pad_e = E_PAD - E
    edges = []
    for ei in (edge_index0, edge_index1):
        s = jnp.concatenate([ei[0], jnp.zeros((pad_e,), jnp.int32)])
        d = jnp.concatenate([ei[1], jnp.full((pad_e,), TRASH, jnp.int32)])
        edges.append((s, d))
    r16 = np.zeros((16, HID), np.float32)
    for h in range(H):
        r16[h, h * DH:(h + 1) * DH] = 1.0
    r16 = jnp.asarray(r16)
    w0p = jnp.pad(mlp_W0, ((0, 0), (0, HID - mlp_W0.shape[1])))
    b0p = _row(jnp.pad(mlp_b0, (0, HID - mlp_b0.shape[0])))
    w1p = jnp.pad(mlp_W1, ((0, HID - mlp_W1.shape[0]), (0, HID - mlp_W1.shape[1])))
    b1p = _row(jnp.pad(mlp_b1, (0, HID - mlp_b1.shape[0])))
    w2p = jnp.pad(mlp_W2, ((0, HID - mlp_W2.shape[0]), (0, HID - mlp_W2.shape[1])))
    b2p = _row(jnp.pad(mlp_b2, (0, HID - mlp_b2.shape[0])))

    fullw = _full((HID, HID))
    fullb = _full((1, HID))
    w256 = _full((HID, 2 * HID))
    b256 = _full((1, 2 * HID))
    shd = jax.ShapeDtypeStruct((N, HID), _f32)

    h0, q1, k1, v1 = pl.pallas_call(
        _embed_qkv_body,
        grid=(GRID,),
        in_specs=[_rows(), _rows(), fullw, fullw, fullb,
                  fullw, fullb, fullw, fullb, fullw, fullb],
        out_specs=[_rows(), _rows(), _rows(), _rows()],
        out_shape=[shd, shd, shd, shd],
    )(x, lap_p, emb_h_W, wl_p, _row(emb_h_b),
      Wq[0], _row(bq[0]), Wk[0], _row(bk[0]), Wv[0], _row(bv[0]))

    num = den = None
    for i in range(2):
        s_e, d_e = edges[i]
        qq, kk, vv = (q1, k1, v1) if i == 0 else (q2, k2, v2)
        sc_e, mx = _sc_score(s_e, d_e, kk, qq)
        gmax = jnp.maximum(jnp.max(mx), 0.0)
        gm_arr = jnp.full((16,), gmax, _f32)
        num = _sc_message(s_e, d_e, sc_e, vv, gm_arr)
        den = num[:, DBASE:DBASE + DROWS].reshape(NCORES, NROWS, 16)

        post_in = [num, den, h0, r16, Wo[i], _row(bo[i]),
                   _row(ln1_g[i]), _row(ln1_b[i]), ffn_W1[i], _row(ffn_b1[i]),
                   ffn_W2[i], _row(ffn_b2[i]), _row(ln2_g[i]), _row(ln2_b[i])]
        post_specs = [_rows((NCORES, RB, HID)), _rows((NCORES, RB, 16)),
                      _rows(), _full((16, HID)), fullw, fullb,
                      fullb, fullb, w256, b256,
                      _full((2 * HID, HID)), fullb, fullb, fullb]
        if i == 0:
            q2, k2, v2 = pl.pallas_call(
                _post_qkv_body,
                grid=(GRID,),
                in_specs=post_specs + [fullw, fullb, fullw, fullb, fullw, fullb],
                out_specs=[_rows(), _rows(), _rows()],
                out_shape=[shd, shd, shd],
            )(*post_in, Wq[1], _row(bq[1]), Wk[1], _row(bk[1]), Wv[1], _row(bv[1]))
        else:
            logits = pl.pallas_call(
                _post_readout_body,
                grid=(GRID,),
                in_specs=post_specs + [fullw, fullb, fullw, fullb, fullw, fullb],
                out_specs=_rows(),
                out_shape=shd,
            )(*post_in, w0p, b0p, w1p, b1p, w2p, b2p)
    return logits[:, :NC_OUT]


# speedup vs baseline: 43.8413x; 1.1895x over previous
import functools

import jax
import jax.numpy as jnp
import numpy as np
from jax import lax
from jax.experimental import pallas as pl
from jax.experimental.pallas import tpu as pltpu
from jax.experimental.pallas import tpu_sc as plsc

N = 10000
E = 320000
HID = 128
H = 8
DH = 16
NC_OUT = 40

NCORES = 2
NSUB = 16
NTILES = NCORES * NSUB
EBS = 128
EB = 32
E_PAD = ((E + NTILES * EBS - 1) // (NTILES * EBS)) * (NTILES * EBS)
PER_TILE = E_PAD // NTILES
NBS = PER_TILE // EBS
NB = PER_TILE // EB
TRASH = N + 8
NROWS = 10016
DBASE = NROWS
DROWS = NROWS // 8
TOTAL_ROWS = 11392
SUBROWS = TOTAL_ROWS // NSUB
RB = 1000
GRID = N // RB

_f32 = jnp.float32


def _mesh():
    return plsc.VectorSubcoreMesh(core_axis_name="c", subcore_axis_name="s")


@functools.partial(
    pl.kernel,
    out_type=[
        jax.ShapeDtypeStruct((E_PAD * 16,), _f32),
        jax.ShapeDtypeStruct((NTILES * 16,), _f32),
    ],
    mesh=_mesh(),
    scratch_types=[
        pltpu.VMEM((2, EBS), jnp.int32),
        pltpu.VMEM((2, EBS), jnp.int32),
        pltpu.VMEM((2, EBS, HID), _f32),
        pltpu.VMEM((2, EBS, HID), _f32),
        pltpu.VMEM((2, EBS * 16), _f32),
        pltpu.VMEM((16,), _f32),
        pltpu.SemaphoreType.DMA,
        pltpu.SemaphoreType.DMA,
        pltpu.SemaphoreType.DMA,
    ],
    compiler_params=pltpu.CompilerParams(needs_layout_passes=False),
)
def _sc_score(s_hbm, d_hbm, k_hbm, q_hbm, sc_out, mx_out,
              sidx, didx, krows, qrows, scb, mxb, semi, semg, semw):
    wid = lax.axis_index("s") * NCORES + lax.axis_index("c")
    base = wid * PER_TILE
    mxb[...] = jnp.zeros((16,), _f32)
    lane = lax.iota(jnp.int32, 16)

    def idx_copies(b, t):
        off = base + b * EBS
        c1 = pltpu.make_async_copy(s_hbm.at[pl.ds(off, EBS)], sidx.at[t], semi)
        c2 = pltpu.make_async_copy(d_hbm.at[pl.ds(off, EBS)], didx.at[t], semi)
        return c1, c2

    def gathers(t):
        c1 = pltpu.make_async_copy(k_hbm.at[sidx.at[t]], krows.at[t], semg)
        c2 = pltpu.make_async_copy(q_hbm.at[didx.at[t]], qrows.at[t], semg)
        return c1, c2

    for cp in idx_copies(0, 0):
        cp.start()
    for cp in idx_copies(0, 0):
        cp.wait()
    for cp in gathers(0):
        cp.start()
    for cp in idx_copies(1, 1):
        cp.start()

    def blk(b, _):
        t = lax.rem(b, 2)
        tn = 1 - t
        for cp in gathers(t):
            cp.wait()

        @pl.when(b + 1 < NBS)
        def _():
            for cp in idx_copies(b + 1, tn):
                cp.wait()
            for cp in gathers(tn):
                cp.start()

        @pl.when(b + 2 < NBS)
        def _():
            for cp in idx_copies(b + 2, t):
                cp.start()

        @pl.when(b >= 2)
        def _():
            pltpu.make_async_copy(
                scb.at[t], sc_out.at[pl.ds(0, EBS * 16)], semw).wait()

        @plsc.parallel_loop(0, EBS, carry=jnp.zeros((16,), _f32))
        def mx_final(e, mx):
            acc = jnp.zeros((16,), _f32)
            for h in range(H):
                kv = krows[t, e, pl.ds(h * DH, DH)]
                qv = qrows[t, e, pl.ds(h * DH, DH)]
                sh = jnp.sum(kv * qv) * 0.25
                acc = jnp.where(lane == h, sh, acc)
            scb[t, pl.ds(e * 16, 16)] = acc
            return jnp.maximum(mx, acc)

        mxb[...] = jnp.maximum(mxb[...], mx_final)
        off = base + b * EBS
        pltpu.make_async_copy(
            scb.at[t], sc_out.at[pl.ds(off * 16, EBS * 16)], semw).start()
        return 0

    lax.fori_loop(0, NBS, blk, 0)
    for _ in range(2):
        pltpu.make_async_copy(
            scb.at[0], sc_out.at[pl.ds(0, EBS * 16)], semw).wait()
    mxb[...] = jnp.full((16,), jnp.max(mxb[...]), _f32)
    pltpu.sync_copy(mxb, mx_out.at[pl.ds(wid * 16, 16)])


@functools.partial(
    pl.kernel,
    out_type=jax.ShapeDtypeStruct((NCORES, TOTAL_ROWS, HID), _f32),
    mesh=_mesh(),
    scratch_types=[
        pltpu.VMEM((2, EB), jnp.int32),
        pltpu.VMEM((2, EB), jnp.int32),
        pltpu.VMEM((2, EB), jnp.int32),
        pltpu.VMEM((2, EB), jnp.int32),
        pltpu.VMEM((2, EB, HID), _f32),
        pltpu.VMEM((2, EB * 16), _f32),
        pltpu.VMEM((2, EB, HID), _f32),
        pltpu.VMEM((2, EB, HID), _f32),
        pltpu.VMEM((16,), _f32),
        pltpu.VMEM_SHARED((TOTAL_ROWS, HID), _f32),
        pltpu.SemaphoreType.DMA,
        pltpu.SemaphoreType.DMA,
        pltpu.SemaphoreType.DMA,
    ],
    compiler_params=pltpu.CompilerParams(needs_layout_passes=False),
)
def _sc_message(s_hbm, d_hbm, sc_hbm, v_hbm, gm_hbm,
                num_out,
                sidx, didx, didx_sc, didx2, vrows, scb, msg, denb, gm,
                num_sp, semi, semg, semw):
    c = lax.axis_index("c")
    s = lax.axis_index("s")
    wid = s * NCORES + c
    base = wid * PER_TILE
    pltpu.sync_copy(gm_hbm, gm)
    z16 = jnp.zeros((16,), _f32)
    lane = lax.iota(jnp.int32, 16)
    mlo = jnp.where(lane < H, 1.0, 0.0).astype(_f32)

    def zrow(i, _):
        for tt in range(2):
            for j in range(HID // 16):
                denb[tt, i, pl.ds(j * 16, 16)] = z16
        return 0

    lax.fori_loop(0, EB, zrow, 0)
    for j in range(SUBROWS // EB):
        pltpu.sync_copy(denb.at[0], num_sp.at[pl.ds(s * SUBROWS + j * EB, EB)])
    rem = SUBROWS % EB
    if rem:
        pltpu.sync_copy(denb.at[0].at[pl.ds(0, rem)],
                        num_sp.at[pl.ds(s * SUBROWS + (SUBROWS // EB) * EB, rem)])
    plsc.subcore_barrier()

    def idx_copies(b, t):
        off = base + b * EB
        c1 = pltpu.make_async_copy(s_hbm.at[pl.ds(off, EB)], sidx.at[t], semi)
        c2 = pltpu.make_async_copy(d_hbm.at[pl.ds(off, EB)], didx.at[t], semi)
        return c1, c2

    def gathers(b, t):
        off = base + b * EB
        c1 = pltpu.make_async_copy(v_hbm.at[sidx.at[t]], vrows.at[t], semg)
        c2 = pltpu.make_async_copy(
            sc_hbm.at[pl.ds(off * 16, EB * 16)], scb.at[t], semg)
        return c1, c2

    def scatter_drains():
        c1 = pltpu.make_async_copy(msg.at[0], num_sp.at[pl.ds(0, EB)], semw)
        c2 = pltpu.make_async_copy(denb.at[0], num_sp.at[pl.ds(0, EB)], semw)
        return c1, c2

    for cp in idx_copies(0, 0):
        cp.start()
    for cp in idx_copies(0, 0):
        cp.wait()
    for cp in gathers(0, 0):
        cp.start()
    for cp in idx_copies(1, 1):
        cp.start()

    def blk(b, _):
        t = lax.rem(b, 2)
        tn = 1 - t
        for cp in gathers(b, t):
            cp.wait()

        @pl.when(b + 1 < NB)
        def _():
            for cp in idx_copies(b + 1, tn):
                cp.wait()
            for cp in gathers(b + 1, tn):
                cp.start()

        for g in range(EB // 16):
            dv = didx[t, pl.ds(g * 16, 16)]
            didx_sc[t, pl.ds(g * 16, 16)] = dv
            didx2[t, pl.ds(g * 16, 16)] = DBASE + lax.shift_right_logical(dv, 3)

        @pl.when(b + 2 < NB)
        def _():
            for cp in idx_copies(b + 2, t):
                cp.start()

        @pl.when(b >= 2)
        def _():
            for cp in scatter_drains():
                cp.wait()

        gmv = gm[...]

        @plsc.parallel_loop(0, EB // 16)
        def _(g):
            dvec = didx_sc[t, pl.ds(g * 16, 16)]
            for e16 in range(16):
                e = g * 16 + e16
                d_e = dvec[e16]
                exr = jnp.exp(scb[t, pl.ds(e * 16, 16)] - gmv)
                for h in range(H):
                    msg[t, e, pl.ds(h * DH, DH)] = exr[h] * vrows[t, e, pl.ds(h * DH, DH)]
                for j in range(HID // 16):
                    denb[t, e, pl.ds(j * 16, 16)] = z16
                doff = (d_e & 7) * 16
                denb[t, e, pl.ds(doff, 16)] = exr * mlo
        pltpu.async_copy(msg.at[t], num_sp.at[didx_sc.at[t]], semw, add=True)
        pltpu.async_copy(denb.at[t], num_sp.at[didx2.at[t]], semw, add=True)
        return 0

    lax.fori_loop(0, NB, blk, 0)
    for _ in range(2):
        for cp in scatter_drains():
            cp.wait()
    plsc.subcore_barrier()
    pltpu.sync_copy(num_sp.at[pl.ds(s * SUBROWS, SUBROWS)],
                    num_out.at[c, pl.ds(s * SUBROWS, SUBROWS)])


def _full(shape):
    return pl.BlockSpec(shape, lambda i: tuple(0 for _ in shape))


def _rows(depth=None):
    if depth is None:
        return pl.BlockSpec((RB, HID), lambda i: (i, 0))
    return pl.BlockSpec(depth, lambda i: (0, i, 0))


def _mm(a, b):
    return jnp.dot(a, b, preferred_element_type=_f32)


def _ln_tc(x, g, b):
    mu = jnp.mean(x, axis=-1, keepdims=True)
    xc = x - mu
    var = jnp.mean(xc * xc, axis=-1, keepdims=True)
    return xc * lax.rsqrt(var + 1e-5) * g + b


def _embed_qkv_body(x_ref, lap_ref, we_ref, wl_ref, be_ref,
                    wq_ref, bq_ref, wk_ref, bk_ref, wv_ref, bv_ref,
                    h0_ref, q_ref, k_ref, v_ref):
    h0 = _mm(x_ref[...], we_ref[...]) + _mm(lap_ref[...], wl_ref[...]) + be_ref[...]
    h0_ref[...] = h0
    q_ref[...] = _mm(h0, wq_ref[...]) + bq_ref[...]
    k_ref[...] = _mm(h0, wk_ref[...]) + bk_ref[...]
    v_ref[...] = _mm(h0, wv_ref[...]) + bv_ref[...]


def _attn_post(num_ref, den_ref, h0_ref, r16_ref, wo_ref, bo_ref,
               g1_ref, b1_ref, w1_ref, c1_ref, w2_ref, c2_ref,
               g2_ref, b2_ref):
    nm = num_ref[0] + num_ref[1]
    dn = jnp.sum(den_ref[...], axis=0) + 1e-9
    den_exp = _mm(dn, r16_ref[...])
    h_attn = nm / den_exp
    h = _mm(h_attn, wo_ref[...]) + bo_ref[...]
    t = _ln_tc(h0_ref[...] + h, g1_ref[...], b1_ref[...])
    h2 = _mm(jnp.maximum(_mm(t, w1_ref[...]) + c1_ref[...], 0.0), w2_ref[...]) + c2_ref[...]
    return _ln_tc(t + h2, g2_ref[...], b2_ref[...])


def _post_qkv_body(num_ref, den_ref, h0_ref, r16_ref, wo_ref, bo_ref,
                   g1_ref, b1_ref, w1_ref, c1_ref, w2_ref, c2_ref,
                   g2_ref, b2_ref,
                   wq_ref, bq_ref, wk_ref, bk_ref, wv_ref, bv_ref,
                   q_ref, k_ref, v_ref):
    hs = _attn_post(num_ref, den_ref, h0_ref, r16_ref, wo_ref, bo_ref,
                    g1_ref, b1_ref, w1_ref, c1_ref, w2_ref, c2_ref,
                    g2_ref, b2_ref)
    q_ref[...] = _mm(h0_ref[...], wq_ref[...]) + bq_ref[...]
    k_ref[...] = _mm(hs, wk_ref[...]) + bk_ref[...]
    v_ref[...] = _mm(hs, wv_ref[...]) + bv_ref[...]


def _post_readout_body(num_ref, den_ref, h0_ref, r16_ref, wo_ref, bo_ref,
                       g1_ref, b1_ref, w1_ref, c1_ref, w2_ref, c2_ref,
                       g2_ref, b2_ref,
                       w0p_ref, b0p_ref, w1p_ref, b1p_ref, w2p_ref, b2p_ref,
                       out_ref):
    hs = _attn_post(num_ref, den_ref, h0_ref, r16_ref, wo_ref, bo_ref,
                    g1_ref, b1_ref, w1_ref, c1_ref, w2_ref, c2_ref,
                    g2_ref, b2_ref)
    a = jnp.maximum(_mm(hs, w0p_ref[...]) + b0p_ref[...], 0.0)
    bmid = jnp.maximum(_mm(a, w1p_ref[...]) + b1p_ref[...], 0.0)
    out_ref[...] = _mm(bmid, w2p_ref[...]) + b2p_ref[...]


def _row(v):
    return v.reshape(1, -1)


def kernel(x, x_lap_pos_enc, edge_index0, edge_index1, node_ids, src_nodes, dst_nodes,
           emb_h_W, emb_h_b, emb_lap_W, emb_lap_b,
           Wq, bq, Wk, bk, Wv, bv, Wo, bo, ln1_g, ln1_b,
           ffn_W1, ffn_b1, ffn_W2, ffn_b2, ln2_g, ln2_b,
           mlp_W0, mlp_b0, mlp_W1, mlp_b1, mlp_W2, mlp_b2):
    lap_p = jnp.pad(x_lap_pos_enc, ((0, 0), (0, HID - x_lap_pos_enc.shape[1])))
    wl_p = jnp.pad(emb_lap_W, ((0, HID - emb_lap_W.shape[0]), (0, 0)))
    pad_e = E_PAD - E
    edges = []
    for ei in (edge_index0, edge_index1):
        s = jnp.concatenate([ei[0], jnp.zeros((pad_e,), jnp.int32)])
        d = jnp.concatenate([ei[1], jnp.full((pad_e,), TRASH, jnp.int32)])
        edges.append((s, d))
    r16 = np.zeros((16, HID), np.float32)
    for h in range(H):
        r16[h, h * DH:(h + 1) * DH] = 1.0
    r16 = jnp.asarray(r16)
    w0p = jnp.pad(mlp_W0, ((0, 0), (0, HID - mlp_W0.shape[1])))
    b0p = _row(jnp.pad(mlp_b0, (0, HID - mlp_b0.shape[0])))
    w1p = jnp.pad(mlp_W1, ((0, HID - mlp_W1.shape[0]), (0, HID - mlp_W1.shape[1])))
    b1p = _row(jnp.pad(mlp_b1, (0, HID - mlp_b1.shape[0])))
    w2p = jnp.pad(mlp_W2, ((0, HID - mlp_W2.shape[0]), (0, HID - mlp_W2.shape[1])))
    b2p = _row(jnp.pad(mlp_b2, (0, HID - mlp_b2.shape[0])))

    fullw = _full((HID, HID))
    fullb = _full((1, HID))
    w256 = _full((HID, 2 * HID))
    b256 = _full((1, 2 * HID))
    shd = jax.ShapeDtypeStruct((N, HID), _f32)

    h0, q1, k1, v1 = pl.pallas_call(
        _embed_qkv_body,
        grid=(GRID,),
        in_specs=[_rows(), _rows(), fullw, fullw, fullb,
                  fullw, fullb, fullw, fullb, fullw, fullb],
        out_specs=[_rows(), _rows(), _rows(), _rows()],
        out_shape=[shd, shd, shd, shd],
    )(x, lap_p, emb_h_W, wl_p, _row(emb_h_b),
      Wq[0], _row(bq[0]), Wk[0], _row(bk[0]), Wv[0], _row(bv[0]))

    num = den = None
    for i in range(2):
        s_e, d_e = edges[i]
        qq, kk, vv = (q1, k1, v1) if i == 0 else (q2, k2, v2)
        sc_e, mx = _sc_score(s_e, d_e, kk, qq)
        gmax = jnp.maximum(jnp.max(mx), 0.0)
        gm_arr = jnp.full((16,), gmax, _f32)
        num = _sc_message(s_e, d_e, sc_e, vv, gm_arr)
        den = num[:, DBASE:DBASE + DROWS].reshape(NCORES, NROWS, 16)

        post_in = [num, den, h0, r16, Wo[i], _row(bo[i]),
                   _row(ln1_g[i]), _row(ln1_b[i]), ffn_W1[i], _row(ffn_b1[i]),
                   ffn_W2[i], _row(ffn_b2[i]), _row(ln2_g[i]), _row(ln2_b[i])]
        post_specs = [_rows((NCORES, RB, HID)), _rows((NCORES, RB, 16)),
                      _rows(), _full((16, HID)), fullw, fullb,
                      fullb, fullb, w256, b256,
                      _full((2 * HID, HID)), fullb, fullb, fullb]
        if i == 0:
            q2, k2, v2 = pl.pallas_call(
                _post_qkv_body,
                grid=(GRID,),
                in_specs=post_specs + [fullw, fullb, fullw, fullb, fullw, fullb],
                out_specs=[_rows(), _rows(), _rows()],
                out_shape=[shd, shd, shd],
            )(*post_in, Wq[1], _row(bq[1]), Wk[1], _row(bk[1]), Wv[1], _row(bv[1]))
        else:
            logits = pl.pallas_call(
                _post_readout_body,
                grid=(GRID,),
                in_specs=post_specs + [fullw, fullb, fullw, fullb, fullw, fullb],
                out_specs=_rows(),
                out_shape=shd,
            )(*post_in, w0p, b0p, w1p, b1p, w2p, b2p)
    return logits[:, :NC_OUT]
